# Initial kernel scaffold; baseline (speedup 1.0000x reference)
import jax
import jax.numpy as jnp
from jax.experimental import pallas as pl


def kernel(x, edge_index, W, b, gamma, beta):
    raise NotImplementedError("write your pallas kernel here")


# speedup vs baseline: 12.9682x; 12.9682x over previous
import functools

import jax
import jax.numpy as jnp
from jax import lax
from jax.experimental import pallas as pl
from jax.experimental.pallas import tpu as pltpu
from jax.experimental.pallas import tpu_sc as plsc

N = 10000
E = 320000
D = 128

NC = 2
NS = 16
NW = NC * NS
K = 128
EPW = E // NW
C = (EPW + K - 1) // K
C = C + (C % 2)
NP = 10240
RPT = NP // NS

_mesh = plsc.VectorSubcoreMesh(core_axis_name="c", subcore_axis_name="s")


NPR = NP // K


@functools.partial(
    pl.kernel,
    out_type=jax.ShapeDtypeStruct((NC, NP), jnp.float32),
    mesh=_mesh,
    scratch_types=[
        pltpu.VMEM((C, K), jnp.int32),
        pltpu.VMEM((NP,), jnp.float32),
        pltpu.VMEM((NS, RPT), jnp.float32),
        pltpu.VMEM((RPT,), jnp.float32),
        pltpu.VMEM_SHARED((NS, NP), jnp.float32),
    ],
    compiler_params=pltpu.CompilerParams(needs_layout_passes=False),
)
def _deg_kernel(dstp_hbm, zflat_hbm, out_hbm, dst_v, deg_v, blk_v, res_v, part_sp):
    c = lax.axis_index("c")
    s = lax.axis_index("s")
    wid = c * NS + s
    pltpu.sync_copy(dstp_hbm.at[wid], dst_v)
    pltpu.sync_copy(zflat_hbm, deg_v)
    ones16 = jnp.full((16,), 1.0, jnp.float32)

    def body(j, carry):
        for q in range(K // 16):
            idx = dst_v[j, pl.ds(q * 16, 16)]
            plsc.addupdate_scatter(deg_v, [idx], ones16)
        return carry

    lax.fori_loop(0, C, body, 0)
    pltpu.sync_copy(deg_v, part_sp.at[s])
    plsc.subcore_barrier()
    for r in range(NS):
        pltpu.sync_copy(part_sp.at[r, pl.ds(s * RPT, RPT)], blk_v.at[r])

    def comb(i, carry):
        tot = jnp.zeros((16,), jnp.float32)
        for r in range(NS):
            tot = tot + blk_v[r, pl.ds(i * 16, 16)]
        res_v[pl.ds(i * 16, 16)] = tot
        return carry

    lax.fori_loop(0, RPT // 16, comb, 0)
    pltpu.sync_copy(res_v, out_hbm.at[c, pl.ds(s * RPT, RPT)])


@functools.partial(
    pl.kernel,
    out_type=jax.ShapeDtypeStruct((NC, NP, D), jnp.float32),
    mesh=_mesh,
    scratch_types=[
        pltpu.VMEM((C, K), jnp.int32),
        pltpu.VMEM((C, K), jnp.int32),
        pltpu.VMEM((K, D), jnp.float32),
        pltpu.VMEM_SHARED((NP, D), jnp.float32),
        pltpu.SemaphoreType.DMA,
    ],
)
def _msg_kernel(g_hbm, srcp_hbm, dstp_hbm, zrows_hbm, out_hbm,
                src_v, dst_v, rows_v, acc_sp, sem):
    c = lax.axis_index("c")
    s = lax.axis_index("s")
    wid = c * NS + s
    pltpu.sync_copy(srcp_hbm.at[wid], src_v)
    pltpu.sync_copy(dstp_hbm.at[wid], dst_v)
    pltpu.sync_copy(zrows_hbm, rows_v)
    for i in range(RPT // K):
        pltpu.sync_copy(rows_v, acc_sp.at[pl.ds(s * RPT + i * K, K)])
    plsc.subcore_barrier()

    def body(j, carry):
        pltpu.async_copy(g_hbm.at[src_v.at[j]], rows_v, sem).wait()
        pltpu.sync_copy(rows_v, acc_sp.at[dst_v.at[j]], add=True)
        return carry

    lax.fori_loop(0, C, body, 0)
    plsc.subcore_barrier()
    for i in range(RPT // K):
        r0 = s * RPT + i * K
        pltpu.sync_copy(acc_sp.at[pl.ds(r0, K)], rows_v)
        pltpu.sync_copy(rows_v, out_hbm.at[c, pl.ds(r0, K)])


def _dense_body(x_ref, gamma_ref, beta_ref, w_ref, degp_ref, g_ref):
    x = x_ref[...]
    mean = jnp.mean(x, axis=0, keepdims=True)
    xc = x - mean
    var = jnp.mean(xc * xc, axis=0, keepdims=True)
    h = xc * lax.rsqrt(var + 1e-5) * gamma_ref[...] + beta_ref[...]
    h = jnp.maximum(h, 0.0)
    hw = jnp.dot(h, w_ref[...], preferred_element_type=jnp.float32)
    deg = degp_ref[0, :N, :] + degp_ref[1, :N, :] + 1.0
    g_ref[...] = hw * lax.rsqrt(deg)


_dense_call = pl.pallas_call(
    _dense_body,
    out_shape=jax.ShapeDtypeStruct((N, D), jnp.float32),
)


def _combine_body(accp_ref, g_ref, degp_ref, b_ref, out_ref):
    deg = degp_ref[0, :N, :] + degp_ref[1, :N, :] + 1.0
    dinv = lax.rsqrt(deg)
    acc = accp_ref[0, :N, :] + accp_ref[1, :N, :] + g_ref[...]
    out_ref[...] = acc * dinv + b_ref[...]


_combine_call = pl.pallas_call(
    _combine_body,
    out_shape=jax.ShapeDtypeStruct((N, D), jnp.float32),
)


@jax.jit
def kernel(x, edge_index, W, b, gamma, beta):
    pad = C * K - EPW
    src = edge_index[0].reshape(NW, EPW)
    dst = edge_index[1].reshape(NW, EPW)
    srcp = jnp.pad(src, ((0, 0), (0, pad)), constant_values=0)
    dstp = jnp.pad(dst, ((0, 0), (0, pad)), constant_values=NP - 1)
    srcp = srcp.reshape(NW, C, K)
    dstp = dstp.reshape(NW, C, K)
    zeros_rows = jnp.zeros((K, D), jnp.float32)
    zeros_flat = jnp.zeros((NP,), jnp.float32)

    degp = _deg_kernel(dstp, zeros_flat).reshape(NC, NP, 1)
    g = _dense_call(x, gamma.reshape(1, D), beta.reshape(1, D), W, degp)
    accp = _msg_kernel(g, srcp, dstp, zeros_rows)
    return _combine_call(accp, g, degp, b.reshape(1, D))


# speedup vs baseline: 14.0914x; 1.0866x over previous
import functools

import jax
import jax.numpy as jnp
from jax import lax
from jax.experimental import pallas as pl
from jax.experimental.pallas import tpu as pltpu
from jax.experimental.pallas import tpu_sc as plsc

N = 10000
E = 320000
D = 128

NC = 2
NS = 16
NW = NC * NS
K = 128
EPW = E // NW
C = (EPW + K - 1) // K
C = C + (C % 2)
NP = 10240
RPT = NP // NS

_mesh = plsc.VectorSubcoreMesh(core_axis_name="c", subcore_axis_name="s")


NPR = NP // K


@functools.partial(
    pl.kernel,
    out_type=jax.ShapeDtypeStruct((NC, NP), jnp.float32),
    mesh=_mesh,
    scratch_types=[
        pltpu.VMEM((C, K), jnp.int32),
        pltpu.VMEM((NP,), jnp.float32),
        pltpu.VMEM((NS, RPT), jnp.float32),
        pltpu.VMEM((RPT,), jnp.float32),
        pltpu.VMEM_SHARED((NS, NP), jnp.float32),
    ],
    compiler_params=pltpu.CompilerParams(needs_layout_passes=False),
)
def _deg_kernel(dstp_hbm, zflat_hbm, out_hbm, dst_v, deg_v, blk_v, res_v, part_sp):
    c = lax.axis_index("c")
    s = lax.axis_index("s")
    wid = c * NS + s
    pltpu.sync_copy(dstp_hbm.at[wid], dst_v)
    pltpu.sync_copy(zflat_hbm, deg_v)
    ones16 = jnp.full((16,), 1.0, jnp.float32)

    def body(j, carry):
        for q in range(K // 16):
            idx = dst_v[j, pl.ds(q * 16, 16)]
            plsc.addupdate_scatter(deg_v, [idx], ones16)
        return carry

    lax.fori_loop(0, C, body, 0)
    pltpu.sync_copy(deg_v, part_sp.at[s])
    plsc.subcore_barrier()
    for r in range(NS):
        pltpu.sync_copy(part_sp.at[r, pl.ds(s * RPT, RPT)], blk_v.at[r])

    def comb(i, carry):
        tot = jnp.zeros((16,), jnp.float32)
        for r in range(NS):
            tot = tot + blk_v[r, pl.ds(i * 16, 16)]
        res_v[pl.ds(i * 16, 16)] = tot
        return carry

    lax.fori_loop(0, RPT // 16, comb, 0)
    pltpu.sync_copy(res_v, out_hbm.at[c, pl.ds(s * RPT, RPT)])


@functools.partial(
    pl.kernel,
    out_type=jax.ShapeDtypeStruct((NC, NP, D), jnp.float32),
    mesh=_mesh,
    scratch_types=[
        pltpu.VMEM((C // 2, K), jnp.int32),
        pltpu.VMEM((C // 2, K), jnp.int32),
        pltpu.VMEM((K, D), jnp.float32),
        pltpu.VMEM((K, D), jnp.float32),
        pltpu.VMEM_SHARED((NP, D), jnp.float32),
        pltpu.SemaphoreType.DMA,
        pltpu.SemaphoreType.DMA,
    ],
)
def _msg_kernel(g_hbm, srcp_hbm, dstp_hbm, zrows_hbm, out_hbm,
                src_v, dst_v, rows0_v, rows1_v, acc_sp, sem0, sem1):
    c = lax.axis_index("c")
    s = lax.axis_index("s")
    wid = c * NS + s
    C2 = C // 2
    pltpu.sync_copy(zrows_hbm, rows0_v)
    for i in range(RPT // K):
        pltpu.sync_copy(rows0_v, acc_sp.at[pl.ds(s * RPT + i * K, K)])
    plsc.subcore_barrier()

    for p in range(2):
        pltpu.sync_copy(srcp_hbm.at[wid, pl.ds(p * C2, C2)], src_v)
        pltpu.sync_copy(dstp_hbm.at[wid, pl.ds(p * C2, C2)], dst_v)
        pltpu.async_copy(g_hbm.at[src_v.at[0]], rows0_v, sem0)

        def body(i, carry):
            j0 = 2 * i
            pltpu.make_async_copy(g_hbm.at[src_v.at[j0]], rows0_v, sem0).wait()
            pltpu.async_copy(g_hbm.at[src_v.at[j0 + 1]], rows1_v, sem1)
            pltpu.sync_copy(rows0_v, acc_sp.at[dst_v.at[j0]], add=True)
            pltpu.make_async_copy(
                g_hbm.at[src_v.at[j0 + 1]], rows1_v, sem1).wait()

            @pl.when(i + 1 < C2 // 2)
            def _():
                pltpu.async_copy(g_hbm.at[src_v.at[j0 + 2]], rows0_v, sem0)

            pltpu.sync_copy(rows1_v, acc_sp.at[dst_v.at[j0 + 1]], add=True)
            return carry

        lax.fori_loop(0, C2 // 2, body, 0)
    plsc.subcore_barrier()
    for i in range(RPT // K):
        r0 = s * RPT + i * K
        pltpu.sync_copy(acc_sp.at[pl.ds(r0, K)], rows0_v)
        pltpu.sync_copy(rows0_v, out_hbm.at[c, pl.ds(r0, K)])


def _dense_body(x_ref, gamma_ref, beta_ref, w_ref, degp_ref, g_ref):
    x = x_ref[...]
    mean = jnp.mean(x, axis=0, keepdims=True)
    xc = x - mean
    var = jnp.mean(xc * xc, axis=0, keepdims=True)
    h = xc * lax.rsqrt(var + 1e-5) * gamma_ref[...] + beta_ref[...]
    h = jnp.maximum(h, 0.0)
    hw = jnp.dot(h, w_ref[...], preferred_element_type=jnp.float32)
    deg = degp_ref[0, :N, :] + degp_ref[1, :N, :] + 1.0
    g_ref[...] = hw * lax.rsqrt(deg)


_dense_call = pl.pallas_call(
    _dense_body,
    out_shape=jax.ShapeDtypeStruct((N, D), jnp.float32),
)


def _combine_body(accp_ref, g_ref, degp_ref, b_ref, out_ref):
    deg = degp_ref[0, :N, :] + degp_ref[1, :N, :] + 1.0
    dinv = lax.rsqrt(deg)
    acc = accp_ref[0, :N, :] + accp_ref[1, :N, :] + g_ref[...]
    out_ref[...] = acc * dinv + b_ref[...]


_combine_call = pl.pallas_call(
    _combine_body,
    out_shape=jax.ShapeDtypeStruct((N, D), jnp.float32),
)


@jax.jit
def kernel(x, edge_index, W, b, gamma, beta):
    pad = C * K - EPW
    src = edge_index[0].reshape(NW, EPW)
    dst = edge_index[1].reshape(NW, EPW)
    srcp = jnp.pad(src, ((0, 0), (0, pad)), constant_values=0)
    dstp = jnp.pad(dst, ((0, 0), (0, pad)), constant_values=NP - 1)
    srcp = srcp.reshape(NW, C, K)
    dstp = dstp.reshape(NW, C, K)
    zeros_rows = jnp.zeros((K, D), jnp.float32)
    zeros_flat = jnp.zeros((NP,), jnp.float32)

    degp = _deg_kernel(dstp, zeros_flat).reshape(NC, NP, 1)
    g = _dense_call(x, gamma.reshape(1, D), beta.reshape(1, D), W, degp)
    accp = _msg_kernel(g, srcp, dstp, zeros_rows)
    return _combine_call(accp, g, degp, b.reshape(1, D))
